# baseline (speedup 1.0000x reference)
import functools

import jax
import jax.numpy as jnp
from jax import lax
from jax.experimental import pallas as pl
from jax.experimental.pallas import tpu as pltpu
from jax.experimental.pallas import tpu_sc as plsc

NC = 2
NS = 16
NW = NC * NS
CHUNK = 128
NBUF = 4
L = 16


@functools.lru_cache(maxsize=None)
def _build_fmt(V, D):
    n_vt = (V + 127) // 128
    per_w = -(-n_vt // NW)
    nbuf = 5
    assert per_w % nbuf == 0
    n_rounds = per_w // nbuf
    Vp = ((V + 127) // 128) * 128

    mesh = plsc.VectorSubcoreMesh(
        core_axis_name="c", subcore_axis_name="s",
        num_cores=NC, num_subcores=NS)

    @functools.partial(
        pl.kernel,
        out_type=jax.ShapeDtypeStruct((Vp, 2 * D), jnp.float32),
        mesh=mesh,
        scratch_types=[
            *[pltpu.VMEM((D, 128), jnp.float32) for _ in range(nbuf)],
            *[pltpu.VMEM((128, 128), jnp.float32) for _ in range(nbuf)],
            *[pltpu.SemaphoreType.DMA for _ in range(2 * nbuf)],
        ],
        compiler_params=pltpu.CompilerParams(
            use_tc_tiling_on_sc=True, needs_layout_passes=False),
    )
    def fmt_kernel(tabT_hbm, out_hbm, *rest):
        ibufs = rest[:nbuf]
        obufs = rest[nbuf:2 * nbuf]
        isems = rest[2 * nbuf:3 * nbuf]
        osems = rest[3 * nbuf:]
        wid = lax.axis_index("s") * NC + lax.axis_index("c")
        t0 = wid * per_w

        lane = jax.lax.iota(jnp.int32, L)
        skew = [jnp.bitwise_and(lane + jnp.int32(o), jnp.int32(L - 1))
                for o in range(L)]

        def fire_in(t, b):
            @pl.when(t < n_vt)
            def _():
                pltpu.async_copy(
                    tabT_hbm.at[:, pl.ds(t * 128, 128)], ibufs[b], isems[b])

        def fire_out(t, b):
            @pl.when(t < n_vt)
            def _():
                pltpu.async_copy(
                    obufs[b], out_hbm.at[pl.ds(t * 128, 128)], osems[b])

        for b in range(nbuf):
            fire_in(t0 + b, b)

        @pl.loop(0, n_rounds)
        def _round(r):
            for b in range(nbuf):
                k = r * nbuf + b
                t = t0 + k

                @pl.when(t < n_vt)
                def _wait_in():
                    pltpu.make_async_copy(
                        tabT_hbm.at[:, pl.ds(t * 128, 128)],
                        ibufs[b], isems[b]).wait()

                @pl.when(jnp.logical_and(r > 0, t - nbuf < n_vt))
                def _wait_out():
                    pltpu.make_async_copy(
                        obufs[b], out_hbm.at[pl.ds((t - nbuf) * 128, 128)],
                        osems[b]).wait()

                src = ibufs[b]
                dst = obufs[b]

                @plsc.parallel_loop(0, 128 // L)
                def _vb(vb):
                    c_vec = lane + vb * L
                    for db in range(D // L):
                        for o in range(L):
                            d_vec = skew[o] + jnp.int32(db * L)
                            vals = plsc.load_gather(src, [d_vec, c_vec])
                            plsc.store_scatter(dst, [c_vec, d_vec], vals)

                fire_out(t, b)

                @pl.when(jnp.logical_and(r < n_rounds - 1,
                                         t + nbuf < n_vt))
                def _refill():
                    pltpu.async_copy(
                        tabT_hbm.at[:, pl.ds((t + nbuf) * 128, 128)],
                        ibufs[b], isems[b])

        for b in range(nbuf):
            t = t0 + (n_rounds - 1) * nbuf + b

            @pl.when(t < n_vt)
            def _drain():
                pltpu.make_async_copy(
                    obufs[b], out_hbm.at[pl.ds(t * 128, 128)],
                    osems[b]).wait()

    return fmt_kernel


@functools.lru_cache(maxsize=None)
def _build(V, D, B, S):
    assert D == 64 and CHUNK == 128
    n_units = B // CHUNK
    n_chunks = n_units // NW
    n_rounds = n_chunks // NBUF
    assert n_chunks % NBUF == 0
    nbt = (B // S) // CHUNK

    mesh = plsc.VectorSubcoreMesh(
        core_axis_name="c", subcore_axis_name="s",
        num_cores=NC, num_subcores=NS)

    @functools.partial(
        pl.kernel,
        out_type=jax.ShapeDtypeStruct((S, 8, nbt, 8, CHUNK), jnp.float32),
        mesh=mesh,
        scratch_types=[
            pltpu.VMEM((n_chunks, CHUNK), jnp.int32),
            *[pltpu.VMEM((CHUNK, 2 * D), jnp.float32) for _ in range(NBUF)],
            *[pltpu.VMEM((8, 8, CHUNK), jnp.float32) for _ in range(NBUF)],
            *[pltpu.SemaphoreType.DMA for _ in range(2 * NBUF)],
        ],
        compiler_params=pltpu.CompilerParams(
            use_tc_tiling_on_sc=True, needs_layout_passes=False),
    )
    def gather_kernel(idx_hbm, tab_hbm, out_hbm, idx_v, *rest):
        gbufs = rest[:NBUF]
        tbufs = rest[NBUF:2 * NBUF]
        gsems = rest[2 * NBUF:3 * NBUF]
        wsems = rest[3 * NBUF:]
        wid = lax.axis_index("s") * NC + lax.axis_index("c")
        chunk0 = wid * n_chunks

        pltpu.sync_copy(idx_hbm.at[pl.ds(chunk0, n_chunks)], idx_v)

        def fire_gather(k, b):
            pltpu.async_copy(tab_hbm.at[idx_v.at[k]], gbufs[b], gsems[b])

        for b in range(NBUF):
            fire_gather(b, b)

        lane = jax.lax.iota(jnp.int32, L)
        skew = [jnp.bitwise_and(lane + jnp.int32(o), jnp.int32(L - 1))
                for o in range(L)]

        @pl.loop(0, n_rounds)
        def _round(r):
            for b in range(NBUF):
                k = r * NBUF + b
                g = chunk0 + k
                s = jax.lax.div(g, nbt)
                bt = jax.lax.rem(g, nbt)
                pltpu.make_async_copy(
                    tab_hbm.at[idx_v.at[k]], gbufs[b], gsems[b]).wait()

                @pl.when(r > 0)
                def _drain_w():
                    pltpu.make_async_copy(
                        tbufs[b], out_hbm.at[s, :, bt], wsems[b]).wait()

                src = gbufs[b]
                dst = tbufs[b]

                @plsc.parallel_loop(0, CHUNK // L)
                def _jg(jg):
                    j_vec = lane + jg * L
                    for o in range(L):
                        s3 = jax.lax.shift_right_logical(
                            skew[o], jnp.int32(3))
                        a7 = jnp.bitwise_and(skew[o], jnp.int32(7))
                        for db in range(D // L):
                            c_vec = skew[o] + jnp.int32(db * L)
                            dt_vec = s3 + jnp.int32(db * 2)
                            vals = plsc.load_gather(src, [j_vec, c_vec])
                            plsc.store_scatter(
                                dst, [dt_vec, a7, j_vec], vals)

                pltpu.async_copy(tbufs[b], out_hbm.at[s, :, bt], wsems[b])

                @pl.when(k + NBUF < n_chunks)
                def _refill():
                    fire_gather(k + NBUF, b)

        for b in range(NBUF):
            k = (n_rounds - 1) * NBUF + b
            g = chunk0 + k
            s = jax.lax.div(g, nbt)
            bt = jax.lax.rem(g, nbt)
            pltpu.make_async_copy(
                tbufs[b], out_hbm.at[s, :, bt], wsems[b]).wait()

    return gather_kernel


def kernel(X, table):
    BATCH, S = X.shape
    V, D = table.shape
    B = BATCH * S
    idx = jnp.transpose(X).reshape(B // CHUNK, CHUNK).astype(jnp.int32)
    tab_p = _build_fmt(V, D)(jnp.transpose(table))
    out5 = _build(V, D, B, S)(idx, tab_p)
    out = jnp.transpose(out5, (2, 4, 0, 1, 3)).reshape(BATCH, S, D)
    return out


# speedup vs baseline: 1.1388x; 1.1388x over previous
import functools

import jax
import jax.numpy as jnp
from jax import lax
from jax.experimental import pallas as pl
from jax.experimental.pallas import tpu as pltpu
from jax.experimental.pallas import tpu_sc as plsc

NC = 2
NS = 16
NW = NC * NS
CHUNK = 128
NBUF = 4
L = 16


@functools.lru_cache(maxsize=None)
def _build(V, D, B, S):
    assert D == 64 and CHUNK == 128
    n_units = B // CHUNK
    n_chunks = n_units // NW
    n_rounds = n_chunks // NBUF
    assert n_chunks % NBUF == 0
    nbt = (B // S) // CHUNK

    mesh = plsc.VectorSubcoreMesh(
        core_axis_name="c", subcore_axis_name="s",
        num_cores=NC, num_subcores=NS)

    @functools.partial(
        pl.kernel,
        out_type=jax.ShapeDtypeStruct((S, 8, nbt, 8, CHUNK), jnp.float32),
        mesh=mesh,
        scratch_types=[
            pltpu.VMEM((n_chunks, CHUNK), jnp.int32),
            *[pltpu.VMEM((CHUNK, 2 * D), jnp.float32) for _ in range(NBUF)],
            *[pltpu.VMEM((8, 8, CHUNK), jnp.float32) for _ in range(NBUF)],
            *[pltpu.SemaphoreType.DMA for _ in range(2 * NBUF)],
        ],
        compiler_params=pltpu.CompilerParams(
            use_tc_tiling_on_sc=False, needs_layout_passes=False),
    )
    def gather_kernel(idx_hbm, tab_hbm, out_hbm, idx_v, *rest):
        gbufs = rest[:NBUF]
        tbufs = rest[NBUF:2 * NBUF]
        gsems = rest[2 * NBUF:3 * NBUF]
        wsems = rest[3 * NBUF:]
        wid = lax.axis_index("s") * NC + lax.axis_index("c")
        chunk0 = wid * n_chunks

        pltpu.sync_copy(idx_hbm.at[pl.ds(chunk0, n_chunks)], idx_v)

        def fire_gather(k, b):
            pltpu.async_copy(tab_hbm.at[idx_v.at[k]], gbufs[b], gsems[b])

        for b in range(NBUF):
            fire_gather(b, b)

        lane = jax.lax.iota(jnp.int32, L)
        skew = [jnp.bitwise_and(lane + jnp.int32(o), jnp.int32(L - 1))
                for o in range(L)]

        @pl.loop(0, n_rounds)
        def _round(r):
            for b in range(NBUF):
                k = r * NBUF + b
                g = chunk0 + k
                s = jax.lax.div(g, nbt)
                bt = jax.lax.rem(g, nbt)
                pltpu.make_async_copy(
                    tab_hbm.at[idx_v.at[k]], gbufs[b], gsems[b]).wait()

                @pl.when(r > 0)
                def _drain_w():
                    pltpu.make_async_copy(
                        tbufs[b], out_hbm.at[s, :, bt], wsems[b]).wait()

                src = gbufs[b]
                dst = tbufs[b]

                @plsc.parallel_loop(0, CHUNK // L)
                def _jg(jg):
                    j_vec = lane + jg * L
                    for o in range(L):
                        s3 = jax.lax.shift_right_logical(
                            skew[o], jnp.int32(3))
                        a7 = jnp.bitwise_and(skew[o], jnp.int32(7))
                        for db in range(D // L):
                            c_vec = skew[o] + jnp.int32(db * L)
                            dt_vec = s3 + jnp.int32(db * 2)
                            vals = plsc.load_gather(src, [j_vec, c_vec])
                            plsc.store_scatter(
                                dst, [dt_vec, a7, j_vec], vals)

                pltpu.async_copy(tbufs[b], out_hbm.at[s, :, bt], wsems[b])

                @pl.when(k + NBUF < n_chunks)
                def _refill():
                    fire_gather(k + NBUF, b)

        for b in range(NBUF):
            k = (n_rounds - 1) * NBUF + b
            g = chunk0 + k
            s = jax.lax.div(g, nbt)
            bt = jax.lax.rem(g, nbt)
            pltpu.make_async_copy(
                tbufs[b], out_hbm.at[s, :, bt], wsems[b]).wait()

    return gather_kernel


def kernel(X, table):
    BATCH, S = X.shape
    V, D = table.shape
    B = BATCH * S
    idx = jnp.transpose(X).reshape(B // CHUNK, CHUNK).astype(jnp.int32)
    tab_p = jnp.pad(table, ((0, 0), (0, D)))
    out5 = _build(V, D, B, S)(idx, tab_p)
    out = jnp.transpose(out5, (2, 4, 0, 1, 3)).reshape(BATCH, S, D)
    return out
